# Initial kernel scaffold; baseline (speedup 1.0000x reference)
import jax
import jax.numpy as jnp
from jax.experimental import pallas as pl


def kernel(x, edge_index, edge_attr, W_kqv, b_kqv, W_edge, b_edge, W_a1, b_a1, W_a2, b_a2, W_out, b_out, bn_gamma, bn_beta, bn_mean, bn_var):
    raise NotImplementedError("write your pallas kernel here")


# speedup vs baseline: 3.7857x; 3.7857x over previous
import functools
import math

import jax
import jax.numpy as jnp
from jax import lax
from jax.experimental import pallas as pl
from jax.experimental.pallas import tpu as pltpu
from jax.experimental.pallas import tpu_sc as plsc

N = 10000
E = 320000
EMB = 128
H = 2
HD = EMB // H
EA = 2

NC = 2
NS = 16
NW = NC * NS
L = 16

EPW = E // NW
CB = 80
KC = 5
C = CB * KC
NCHUNK = EPW // C
CB2 = 40
KC2 = 5
C2 = CB2 * KC2
NCHUNK2 = EPW // C2
RPT = 624
RW = EMB + L
SR = 64

_f32 = jnp.float32


def _stage1_body(x_ref, wkqv_ref, bkqv_ref, wk_ref, wq_ref,
                 kw_ref, qw_ref, v_ref):
    x = x_ref[...]
    kqv = lax.dot_general(x, wkqv_ref[...], (((1,), (1,)), ((), ())),
                          preferred_element_type=_f32) + bkqv_ref[...]
    q = kqv[:, :EMB]
    k = kqv[:, EMB:2 * EMB] * (1.0 / math.sqrt(HD))
    v_ref[...] = kqv[:, 2 * EMB:]
    kw_ref[...] = jnp.concatenate(
        [lax.dot_general(k[:, h * HD:(h + 1) * HD], wk_ref[...],
                         (((1,), (1,)), ((), ())), preferred_element_type=_f32)
         for h in range(H)], axis=1)
    qw_ref[...] = jnp.concatenate(
        [lax.dot_general(q[:, h * HD:(h + 1) * HD], wq_ref[...],
                         (((1,), (1,)), ((), ())), preferred_element_type=_f32)
         for h in range(H)], axis=1)


def _stage1(x, W_kqv, b_kqv2, Wk, Wq, interpret=False):
    BN_ = 1000
    grid = (N // BN_,)
    return pl.pallas_call(
        _stage1_body,
        grid=grid,
        in_specs=[
            pl.BlockSpec((BN_, EMB), lambda i: (i, 0)),
            pl.BlockSpec((3 * EMB, EMB), lambda i: (0, 0)),
            pl.BlockSpec((1, 3 * EMB), lambda i: (0, 0)),
            pl.BlockSpec((HD, HD), lambda i: (0, 0)),
            pl.BlockSpec((HD, HD), lambda i: (0, 0)),
        ],
        out_specs=[
            pl.BlockSpec((BN_, EMB), lambda i: (i, 0)),
            pl.BlockSpec((BN_, EMB), lambda i: (i, 0)),
            pl.BlockSpec((BN_, EMB), lambda i: (i, 0)),
        ],
        out_shape=[
            jax.ShapeDtypeStruct((N, EMB), _f32),
            jax.ShapeDtypeStruct((N, EMB), _f32),
            jax.ShapeDtypeStruct((N, EMB), _f32),
        ],
        interpret=interpret,
    )(x, W_kqv, b_kqv2, Wk, Wq)


def _sc_pass1(src1, dst1, kW, qW, interpret=False):
    mesh = plsc.VectorSubcoreMesh(core_axis_name="c", subcore_axis_name="s")

    @functools.partial(
        pl.kernel,
        out_type=jax.ShapeDtypeStruct((E // CB, CB, EMB), _f32),
        mesh=mesh,
        scratch_types=[
            pltpu.VMEM((C,), jnp.int32),
            pltpu.VMEM((C,), jnp.int32),
            pltpu.VMEM((KC, CB, EMB), _f32),
            pltpu.VMEM((KC, CB, EMB), _f32),
            pltpu.SemaphoreType.DMA,
        ],
        interpret=interpret,
    )
    def body(src_hbm, dst_hbm, kw_hbm, qw_hbm, g_hbm,
             sidx, didx, krows, qrows, sem):
        cid = lax.axis_index("c")
        sid = lax.axis_index("s")
        wid = sid * NC + cid
        rows_pw = EPW // CB

        def chunk(i, carry):
            row0 = wid * rows_pw + i * KC
            off = row0 * CB
            pltpu.sync_copy(src_hbm.at[pl.ds(off, C)], sidx)
            pltpu.sync_copy(dst_hbm.at[pl.ds(off, C)], didx)
            cps = []
            for j in range(KC):
                cps.append(pltpu.async_copy(
                    kw_hbm.at[sidx.at[pl.ds(j * CB, CB)]], krows.at[j], sem))
                cps.append(pltpu.async_copy(
                    qw_hbm.at[didx.at[pl.ds(j * CB, CB)]], qrows.at[j], sem))
            for cp in cps:
                cp.wait()

            def add_rows(e, c2):
                for j in range(KC):
                    for t in range(EMB // L):
                        sl = pl.ds(t * L, L)
                        krows[j, e, sl] = krows[j, e, sl] + qrows[j, e, sl]
                return c2

            lax.fori_loop(0, CB, add_rows, 0, unroll=False)
            pltpu.sync_copy(krows, g_hbm.at[pl.ds(row0, KC)])
            return carry

        lax.fori_loop(0, NCHUNK, chunk, 0, unroll=False)

    return body(src1, dst1, kW, qW)


def _stage2_body(g_ref, ea_ref, wedge_ref, bedge_ref, we_ref, ba1_ref,
                 wa2_ref, ba2_ref, ex_ref):
    ea = lax.dot_general(ea_ref[...], wedge_ref[...], (((1,), (1,)), ((), ())),
                         preferred_element_type=_f32) + bedge_ref[...]
    r = jnp.maximum(ea, 0.0)
    g = g_ref[...]
    cols = []
    for h in range(H):
        z = (g[:, h * HD:(h + 1) * HD]
             + lax.dot_general(r[:, h * HD:(h + 1) * HD], we_ref[...],
                               (((1,), (1,)), ((), ())),
                               preferred_element_type=_f32)
             + ba1_ref[...])
        z = jnp.maximum(z, 0.0)
        a_h = jnp.sum(z * wa2_ref[...], axis=1, keepdims=True) + ba2_ref[...]
        cols.append(jnp.exp(a_h))
    b = cols[0].shape[0]
    cols.append(jnp.ones((b, 1), _f32))
    cols.append(jnp.zeros((b, L - H - 1), _f32))
    ex_ref[...] = jnp.concatenate(cols, axis=1)


def _stage2(g, edge_attr, W_edge, b_edge2, We, b_a12, W_a2, b_a22,
            interpret=False):
    BE = 4000
    grid = (E // BE,)
    return pl.pallas_call(
        _stage2_body,
        grid=grid,
        in_specs=[
            pl.BlockSpec((BE, EMB), lambda i: (i, 0)),
            pl.BlockSpec((BE, EA), lambda i: (i, 0)),
            pl.BlockSpec((EMB, EA), lambda i: (0, 0)),
            pl.BlockSpec((1, EMB), lambda i: (0, 0)),
            pl.BlockSpec((HD, HD), lambda i: (0, 0)),
            pl.BlockSpec((1, HD), lambda i: (0, 0)),
            pl.BlockSpec((1, HD), lambda i: (0, 0)),
            pl.BlockSpec((1, 1), lambda i: (0, 0)),
        ],
        out_specs=pl.BlockSpec((BE, L), lambda i: (i, 0)),
        out_shape=jax.ShapeDtypeStruct((E, L), _f32),
        interpret=interpret,
    )(g, edge_attr, W_edge, b_edge2, We, b_a12, W_a2, b_a22)


def _sc_pass2(src1, dst1, ex2, v, interpret=False):
    mesh = plsc.VectorSubcoreMesh(core_axis_name="c", subcore_axis_name="s")

    @functools.partial(
        pl.kernel,
        out_type=[
            jax.ShapeDtypeStruct((NC * N, EMB), _f32),
            jax.ShapeDtypeStruct((NC * N, EMB), _f32),
        ],
        mesh=mesh,
        scratch_types=[
            pltpu.VMEM((C2,), jnp.int32),
            pltpu.VMEM((KC2, CB2), jnp.int32),
            pltpu.VMEM((CB2, L), _f32),
            pltpu.VMEM((KC2, CB2, EMB), _f32),
            pltpu.VMEM_SHARED((N, EMB), _f32),
            pltpu.SemaphoreType.DMA,
        ],
        interpret=interpret,
    )
    def body(src_hbm, dst_hbm, ex_hbm, v_hbm, zrow_hbm, u_hbm, s_hbm,
             sidx, didx, exb, vrows, u_sh, sem):
        cid = lax.axis_index("c")
        sid = lax.axis_index("s")
        wid = sid * NC + cid
        rows_pw = EPW // CB2

        r0 = sid * RPT
        t0 = jnp.where(sid == 0, NS * RPT, r0)
        tail = N - NS * RPT

        def zero_acc():
            pltpu.sync_copy(zrow_hbm.at[pl.ds(r0, RPT)],
                            u_sh.at[pl.ds(r0, RPT)])
            pltpu.sync_copy(zrow_hbm.at[pl.ds(t0, tail)],
                            u_sh.at[pl.ds(t0, tail)])

        def read_acc(out_hbm):
            pltpu.sync_copy(u_sh.at[pl.ds(r0, RPT)],
                            out_hbm.at[pl.ds(cid * N + r0, RPT)])
            pltpu.sync_copy(u_sh.at[pl.ds(t0, tail)],
                            out_hbm.at[pl.ds(cid * N + t0, tail)])

        def load_didx(off):
            for j in range(KC2):
                pltpu.sync_copy(dst_hbm.at[pl.ds(off + j * CB2, CB2)],
                                didx.at[j])

        zero_acc()
        plsc.subcore_barrier()

        def chunk_a(i, carry):
            off = (wid * rows_pw + i * KC2) * CB2
            pltpu.sync_copy(src_hbm.at[pl.ds(off, C2)], sidx)
            load_didx(off)
            cps = [pltpu.async_copy(
                       v_hbm.at[sidx.at[pl.ds(j * CB2, CB2)]], vrows.at[j],
                       sem)
                   for j in range(KC2)]
            for cp in cps:
                cp.wait()
            for j in range(KC2):
                pltpu.sync_copy(ex_hbm.at[pl.ds(off + j * CB2, CB2)], exb)

                def scale_rows(e, c2):
                    s = exb[e, :]
                    e0v = jnp.broadcast_to(s[0], (L,))
                    e1v = jnp.broadcast_to(s[1], (L,))
                    for t in range(HD // L):
                        sl = pl.ds(t * L, L)
                        vrows[j, e, sl] = vrows[j, e, sl] * e0v
                    for t in range(HD // L, EMB // L):
                        sl = pl.ds(t * L, L)
                        vrows[j, e, sl] = vrows[j, e, sl] * e1v
                    return c2

                lax.fori_loop(0, CB2, scale_rows, 0, unroll=False)
                pltpu.sync_copy(vrows.at[j], u_sh.at[didx.at[j]], add=True)
            return carry

        lax.fori_loop(0, NCHUNK2, chunk_a, 0, unroll=False)
        plsc.subcore_barrier()
        read_acc(u_hbm)
        plsc.subcore_barrier()

        zero_acc()
        zero16 = jnp.zeros((L,), _f32)

        def zero_vrows(e, c2):
            for j in range(KC2):
                for t in range(EMB // L):
                    vrows[j, e, pl.ds(t * L, L)] = zero16
            return c2

        lax.fori_loop(0, CB2, zero_vrows, 0, unroll=False)
        plsc.subcore_barrier()

        def chunk_b(i, carry):
            off = (wid * rows_pw + i * KC2) * CB2
            load_didx(off)
            for j in range(KC2):
                pltpu.sync_copy(ex_hbm.at[pl.ds(off + j * CB2, CB2)], exb)

                def stat_rows(e, c2):
                    vrows[j, e, pl.ds(0, L)] = exb[e, :]
                    return c2

                lax.fori_loop(0, CB2, stat_rows, 0, unroll=False)
                pltpu.sync_copy(vrows.at[j], u_sh.at[didx.at[j]], add=True)
            return carry

        lax.fori_loop(0, NCHUNK2, chunk_b, 0, unroll=False)
        plsc.subcore_barrier()
        read_acc(s_hbm)

    zrow = jnp.zeros((N, EMB), _f32)
    return body(src1, dst1, ex2, v, zrow)


def _stage3_body(u0_ref, u1_ref, s0_ref, s1_ref, x_ref, wout_ref, bout_ref,
                 gam_ref, bet_ref, mu_ref, var_ref, out_ref):
    um = u0_ref[...] + u1_ref[...]
    us = s0_ref[...] + s1_ref[...]
    d0 = us[:, 0:1] + 1e-16
    d1 = us[:, 1:2] + 1e-16
    indeg = us[:, 2:3]
    aggp = jnp.concatenate([um[:, :HD] / d0, um[:, HD:] / d1], axis=1)
    agg = lax.dot_general(aggp, wout_ref[...], (((1,), (1,)), ((), ())),
                          preferred_element_type=_f32) + indeg * bout_ref[...]
    o = agg + x_ref[...]
    o = (o - mu_ref[...]) * lax.rsqrt(var_ref[...] + 1e-5) * gam_ref[...] \
        + bet_ref[...]
    out_ref[...] = o


def _stage3(U0, U1, S0, S1, x, W_out, b_out2, gam2, bet2, mu2, var2,
            interpret=False):
    BN_ = 1000
    grid = (N // BN_,)
    return pl.pallas_call(
        _stage3_body,
        grid=grid,
        in_specs=[
            pl.BlockSpec((BN_, EMB), lambda i: (i, 0)),
            pl.BlockSpec((BN_, EMB), lambda i: (i, 0)),
            pl.BlockSpec((BN_, L), lambda i: (i, 0)),
            pl.BlockSpec((BN_, L), lambda i: (i, 0)),
            pl.BlockSpec((BN_, EMB), lambda i: (i, 0)),
            pl.BlockSpec((EMB, EMB), lambda i: (0, 0)),
            pl.BlockSpec((1, EMB), lambda i: (0, 0)),
            pl.BlockSpec((1, EMB), lambda i: (0, 0)),
            pl.BlockSpec((1, EMB), lambda i: (0, 0)),
            pl.BlockSpec((1, EMB), lambda i: (0, 0)),
            pl.BlockSpec((1, EMB), lambda i: (0, 0)),
        ],
        out_specs=pl.BlockSpec((BN_, EMB), lambda i: (i, 0)),
        out_shape=jax.ShapeDtypeStruct((N, EMB), _f32),
        interpret=interpret,
    )(U0, U1, S0, S1, x, W_out, b_out2, gam2, bet2, mu2, var2)


def kernel(x, edge_index, edge_attr, W_kqv, b_kqv, W_edge, b_edge,
           W_a1, b_a1, W_a2, b_a2, W_out, b_out,
           bn_gamma, bn_beta, bn_mean, bn_var):
    src1 = edge_index[0]
    dst1 = edge_index[1]
    Wk = W_a1[:, :HD]
    Wq = W_a1[:, HD:2 * HD]
    We = W_a1[:, 2 * HD:]

    kW, qW, v = _stage1(x, W_kqv, b_kqv.reshape(1, -1), Wk, Wq)
    g3 = _sc_pass1(src1, dst1, kW, qW)
    ex = _stage2(g3.reshape(E, EMB), edge_attr, W_edge,
                 b_edge.reshape(1, -1), We, b_a1.reshape(1, -1),
                 W_a2, b_a2.reshape(1, 1))
    Um, Us = _sc_pass2(src1, dst1, ex, v)
    out = _stage3(Um[:N], Um[N:], Us[:N, :L], Us[N:, :L],
                  x, W_out, b_out.reshape(1, -1),
                  bn_gamma.reshape(1, -1), bn_beta.reshape(1, -1),
                  bn_mean.reshape(1, -1), bn_var.reshape(1, -1))
    return out


# speedup vs baseline: 5.5382x; 1.4629x over previous
import functools
import math

import jax
import jax.numpy as jnp
from jax import lax
from jax.experimental import pallas as pl
from jax.experimental.pallas import tpu as pltpu
from jax.experimental.pallas import tpu_sc as plsc

N = 10000
E = 320000
EMB = 128
H = 2
HD = EMB // H
EA = 2

NC = 2
NS = 16
NW = NC * NS
L = 16

EPW = E // NW
CB = 80
KC = 5
C = CB * KC
NCHUNK = EPW // C
CB2 = 40
KC2 = 5
C2 = CB2 * KC2
NCHUNK2 = EPW // C2
RPT = 624
RW = EMB + L
SR = 64

_f32 = jnp.float32


def _stage1_body(x_ref, wkqv_ref, bkqv_ref, wk_ref, wq_ref,
                 kw_ref, qw_ref, v_ref):
    x = x_ref[...]
    kqv = lax.dot_general(x, wkqv_ref[...], (((1,), (1,)), ((), ())),
                          preferred_element_type=_f32) + bkqv_ref[...]
    q = kqv[:, :EMB]
    k = kqv[:, EMB:2 * EMB] * (1.0 / math.sqrt(HD))
    v_ref[...] = kqv[:, 2 * EMB:]
    kw_ref[...] = jnp.concatenate(
        [lax.dot_general(k[:, h * HD:(h + 1) * HD], wk_ref[...],
                         (((1,), (1,)), ((), ())), preferred_element_type=_f32)
         for h in range(H)], axis=1)
    qw_ref[...] = jnp.concatenate(
        [lax.dot_general(q[:, h * HD:(h + 1) * HD], wq_ref[...],
                         (((1,), (1,)), ((), ())), preferred_element_type=_f32)
         for h in range(H)], axis=1)


def _stage1(x, W_kqv, b_kqv2, Wk, Wq, interpret=False):
    BN_ = 1000
    grid = (N // BN_,)
    return pl.pallas_call(
        _stage1_body,
        grid=grid,
        in_specs=[
            pl.BlockSpec((BN_, EMB), lambda i: (i, 0)),
            pl.BlockSpec((3 * EMB, EMB), lambda i: (0, 0)),
            pl.BlockSpec((1, 3 * EMB), lambda i: (0, 0)),
            pl.BlockSpec((HD, HD), lambda i: (0, 0)),
            pl.BlockSpec((HD, HD), lambda i: (0, 0)),
        ],
        out_specs=[
            pl.BlockSpec((BN_, EMB), lambda i: (i, 0)),
            pl.BlockSpec((BN_, EMB), lambda i: (i, 0)),
            pl.BlockSpec((BN_, EMB), lambda i: (i, 0)),
        ],
        out_shape=[
            jax.ShapeDtypeStruct((N, EMB), _f32),
            jax.ShapeDtypeStruct((N, EMB), _f32),
            jax.ShapeDtypeStruct((N, EMB), _f32),
        ],
        interpret=interpret,
    )(x, W_kqv, b_kqv2, Wk, Wq)


def _sc_pass1(src1, dst1, kW, qW, interpret=False):
    mesh = plsc.VectorSubcoreMesh(core_axis_name="c", subcore_axis_name="s")

    @functools.partial(
        pl.kernel,
        out_type=jax.ShapeDtypeStruct((E // CB, CB, EMB), _f32),
        mesh=mesh,
        scratch_types=[
            pltpu.VMEM((C,), jnp.int32),
            pltpu.VMEM((C,), jnp.int32),
            pltpu.VMEM((KC, CB, EMB), _f32),
            pltpu.VMEM((KC, CB, EMB), _f32),
            pltpu.SemaphoreType.DMA,
        ],
        interpret=interpret,
    )
    def body(src_hbm, dst_hbm, kw_hbm, qw_hbm, g_hbm,
             sidx, didx, krows, qrows, sem):
        cid = lax.axis_index("c")
        sid = lax.axis_index("s")
        wid = sid * NC + cid
        rows_pw = EPW // CB

        def chunk(i, carry):
            row0 = wid * rows_pw + i * KC
            off = row0 * CB
            pltpu.sync_copy(src_hbm.at[pl.ds(off, C)], sidx)
            pltpu.sync_copy(dst_hbm.at[pl.ds(off, C)], didx)
            cps = []
            for j in range(KC):
                cps.append(pltpu.async_copy(
                    kw_hbm.at[sidx.at[pl.ds(j * CB, CB)]], krows.at[j], sem))
                cps.append(pltpu.async_copy(
                    qw_hbm.at[didx.at[pl.ds(j * CB, CB)]], qrows.at[j], sem))
            for cp in cps:
                cp.wait()

            def add_rows(e, c2):
                for j in range(KC):
                    for t in range(EMB // L):
                        sl = pl.ds(t * L, L)
                        krows[j, e, sl] = krows[j, e, sl] + qrows[j, e, sl]
                return c2

            lax.fori_loop(0, CB, add_rows, 0, unroll=False)
            pltpu.sync_copy(krows, g_hbm.at[pl.ds(row0, KC)])
            return carry

        lax.fori_loop(0, NCHUNK, chunk, 0, unroll=False)

    return body(src1, dst1, kW, qW)


def _stage2_body(g_ref, ea_ref, wedge_ref, bedge_ref, we_ref, ba1_ref,
                 wa2_ref, ba2_ref, ex_ref):
    ea = lax.dot_general(ea_ref[...], wedge_ref[...], (((1,), (1,)), ((), ())),
                         preferred_element_type=_f32) + bedge_ref[...]
    r = jnp.maximum(ea, 0.0)
    g = g_ref[...]
    cols = []
    for h in range(H):
        z = (g[:, h * HD:(h + 1) * HD]
             + lax.dot_general(r[:, h * HD:(h + 1) * HD], we_ref[...],
                               (((1,), (1,)), ((), ())),
                               preferred_element_type=_f32)
             + ba1_ref[...])
        z = jnp.maximum(z, 0.0)
        a_h = jnp.sum(z * wa2_ref[...], axis=1, keepdims=True) + ba2_ref[...]
        cols.append(jnp.exp(a_h))
    b = cols[0].shape[0]
    cols.append(jnp.ones((b, 1), _f32))
    cols.append(jnp.zeros((b, L - H - 1), _f32))
    ex_ref[...] = jnp.concatenate(cols, axis=1)


def _stage2(g, edge_attr, W_edge, b_edge2, We, b_a12, W_a2, b_a22,
            interpret=False):
    BE = 4000
    grid = (E // BE,)
    return pl.pallas_call(
        _stage2_body,
        grid=grid,
        in_specs=[
            pl.BlockSpec((BE, EMB), lambda i: (i, 0)),
            pl.BlockSpec((BE, EA), lambda i: (i, 0)),
            pl.BlockSpec((EMB, EA), lambda i: (0, 0)),
            pl.BlockSpec((1, EMB), lambda i: (0, 0)),
            pl.BlockSpec((HD, HD), lambda i: (0, 0)),
            pl.BlockSpec((1, HD), lambda i: (0, 0)),
            pl.BlockSpec((1, HD), lambda i: (0, 0)),
            pl.BlockSpec((1, 1), lambda i: (0, 0)),
        ],
        out_specs=pl.BlockSpec((BE, L), lambda i: (i, 0)),
        out_shape=jax.ShapeDtypeStruct((E, L), _f32),
        interpret=interpret,
    )(g, edge_attr, W_edge, b_edge2, We, b_a12, W_a2, b_a22)


def _sc_pass2(src1, dst1, ex2, v, interpret=False):
    mesh = plsc.VectorSubcoreMesh(core_axis_name="c", subcore_axis_name="s")

    @functools.partial(
        pl.kernel,
        out_type=[
            jax.ShapeDtypeStruct((NC * N, EMB), _f32),
            jax.ShapeDtypeStruct((NC * N, EMB), _f32),
        ],
        mesh=mesh,
        scratch_types=[
            pltpu.VMEM((C2,), jnp.int32),
            pltpu.VMEM((KC2, CB2), jnp.int32),
            pltpu.VMEM((2, CB2, L), _f32),
            pltpu.VMEM((KC2, CB2, EMB), _f32),
            pltpu.VMEM_SHARED((N, EMB), _f32),
            pltpu.SemaphoreType.DMA,
            pltpu.SemaphoreType.DMA,
            pltpu.SemaphoreType.DMA,
            pltpu.SemaphoreType.DMA,
            pltpu.SemaphoreType.DMA,
        ],
        interpret=interpret,
    )
    def body(src_hbm, dst_hbm, ex_hbm, v_hbm, zrow_hbm, u_hbm, s_hbm,
             sidx, didx, exb, vrows, u_sh, semi, semg, seme0, seme1, sems):
        cid = lax.axis_index("c")
        sid = lax.axis_index("s")
        wid = sid * NC + cid
        rows_pw = EPW // CB2

        r0 = sid * RPT
        t0 = jnp.where(sid == 0, NS * RPT, r0)
        tail = N - NS * RPT

        def zero_acc():
            pltpu.sync_copy(zrow_hbm.at[pl.ds(r0, RPT)],
                            u_sh.at[pl.ds(r0, RPT)])
            pltpu.sync_copy(zrow_hbm.at[pl.ds(t0, tail)],
                            u_sh.at[pl.ds(t0, tail)])

        def read_acc(out_hbm):
            pltpu.sync_copy(u_sh.at[pl.ds(r0, RPT)],
                            out_hbm.at[pl.ds(cid * N + r0, RPT)])
            pltpu.sync_copy(u_sh.at[pl.ds(t0, tail)],
                            out_hbm.at[pl.ds(cid * N + t0, tail)])

        def load_idx_async(off, with_src):
            cps = []
            if with_src:
                cps.append(pltpu.async_copy(src_hbm.at[pl.ds(off, C2)],
                                            sidx, semi))
            for j in range(KC2):
                cps.append(pltpu.async_copy(
                    dst_hbm.at[pl.ds(off + j * CB2, CB2)], didx.at[j], semi))
            return cps

        seme = [seme0, seme1]

        def fire_exb(off, j):
            return pltpu.async_copy(
                ex_hbm.at[pl.ds(off + j * CB2, CB2)], exb.at[j % 2],
                seme[j % 2])

        zero_acc()
        plsc.subcore_barrier()

        def chunk_a(i, carry):
            off = (wid * rows_pw + i * KC2) * CB2
            cpi = load_idx_async(off, with_src=True)
            cpe = {0: fire_exb(off, 0)}
            for cp in cpi:
                cp.wait()
            cpg = [pltpu.async_copy(
                       v_hbm.at[sidx.at[pl.ds(j * CB2, CB2)]], vrows.at[j],
                       semg)
                   for j in range(KC2)]
            for cp in cpg:
                cp.wait()
            cps = []
            for j in range(KC2):
                if j + 1 < KC2:
                    cpe[j + 1] = fire_exb(off, j + 1)
                cpe[j].wait()
                b = j % 2

                def scale_rows(e, c2):
                    s = exb[b, e, :]
                    e0v = jnp.broadcast_to(s[0], (L,))
                    e1v = jnp.broadcast_to(s[1], (L,))
                    for t in range(HD // L):
                        sl = pl.ds(t * L, L)
                        vrows[j, e, sl] = vrows[j, e, sl] * e0v
                    for t in range(HD // L, EMB // L):
                        sl = pl.ds(t * L, L)
                        vrows[j, e, sl] = vrows[j, e, sl] * e1v
                    return c2

                lax.fori_loop(0, CB2, scale_rows, 0, unroll=False)
                cps.append(pltpu.async_copy(vrows.at[j],
                                            u_sh.at[didx.at[j]], sems,
                                            add=True))
            for cp in cps:
                cp.wait()
            return carry

        lax.fori_loop(0, NCHUNK2, chunk_a, 0, unroll=False)
        plsc.subcore_barrier()
        read_acc(u_hbm)
        plsc.subcore_barrier()

        zero_acc()
        zero16 = jnp.zeros((L,), _f32)

        def zero_vrows(e, c2):
            for j in range(KC2):
                for t in range(EMB // L):
                    vrows[j, e, pl.ds(t * L, L)] = zero16
            return c2

        lax.fori_loop(0, CB2, zero_vrows, 0, unroll=False)
        plsc.subcore_barrier()

        def chunk_b(i, carry):
            off = (wid * rows_pw + i * KC2) * CB2
            cpi = load_idx_async(off, with_src=False)
            cpe = {0: fire_exb(off, 0)}
            for cp in cpi:
                cp.wait()
            cps = []
            for j in range(KC2):
                if j + 1 < KC2:
                    cpe[j + 1] = fire_exb(off, j + 1)
                cpe[j].wait()
                b = j % 2

                def stat_rows(e, c2):
                    vrows[j, e, pl.ds(0, L)] = exb[b, e, :]
                    return c2

                lax.fori_loop(0, CB2, stat_rows, 0, unroll=False)
                cps.append(pltpu.async_copy(vrows.at[j],
                                            u_sh.at[didx.at[j]], sems,
                                            add=True))
            for cp in cps:
                cp.wait()
            return carry

        lax.fori_loop(0, NCHUNK2, chunk_b, 0, unroll=False)
        plsc.subcore_barrier()
        read_acc(s_hbm)

    zrow = jnp.zeros((N, EMB), _f32)
    return body(src1, dst1, ex2, v, zrow)


def _stage3_body(u0_ref, u1_ref, s0_ref, s1_ref, x_ref, wout_ref, bout_ref,
                 gam_ref, bet_ref, mu_ref, var_ref, out_ref):
    um = u0_ref[...] + u1_ref[...]
    us = s0_ref[...] + s1_ref[...]
    d0 = us[:, 0:1] + 1e-16
    d1 = us[:, 1:2] + 1e-16
    indeg = us[:, 2:3]
    aggp = jnp.concatenate([um[:, :HD] / d0, um[:, HD:] / d1], axis=1)
    agg = lax.dot_general(aggp, wout_ref[...], (((1,), (1,)), ((), ())),
                          preferred_element_type=_f32) + indeg * bout_ref[...]
    o = agg + x_ref[...]
    o = (o - mu_ref[...]) * lax.rsqrt(var_ref[...] + 1e-5) * gam_ref[...] \
        + bet_ref[...]
    out_ref[...] = o


def _stage3(U0, U1, S0, S1, x, W_out, b_out2, gam2, bet2, mu2, var2,
            interpret=False):
    BN_ = 1000
    grid = (N // BN_,)
    return pl.pallas_call(
        _stage3_body,
        grid=grid,
        in_specs=[
            pl.BlockSpec((BN_, EMB), lambda i: (i, 0)),
            pl.BlockSpec((BN_, EMB), lambda i: (i, 0)),
            pl.BlockSpec((BN_, L), lambda i: (i, 0)),
            pl.BlockSpec((BN_, L), lambda i: (i, 0)),
            pl.BlockSpec((BN_, EMB), lambda i: (i, 0)),
            pl.BlockSpec((EMB, EMB), lambda i: (0, 0)),
            pl.BlockSpec((1, EMB), lambda i: (0, 0)),
            pl.BlockSpec((1, EMB), lambda i: (0, 0)),
            pl.BlockSpec((1, EMB), lambda i: (0, 0)),
            pl.BlockSpec((1, EMB), lambda i: (0, 0)),
            pl.BlockSpec((1, EMB), lambda i: (0, 0)),
        ],
        out_specs=pl.BlockSpec((BN_, EMB), lambda i: (i, 0)),
        out_shape=jax.ShapeDtypeStruct((N, EMB), _f32),
        interpret=interpret,
    )(U0, U1, S0, S1, x, W_out, b_out2, gam2, bet2, mu2, var2)


def kernel(x, edge_index, edge_attr, W_kqv, b_kqv, W_edge, b_edge,
           W_a1, b_a1, W_a2, b_a2, W_out, b_out,
           bn_gamma, bn_beta, bn_mean, bn_var):
    src1 = edge_index[0]
    dst1 = edge_index[1]
    Wk = W_a1[:, :HD]
    Wq = W_a1[:, HD:2 * HD]
    We = W_a1[:, 2 * HD:]

    kW, qW, v = _stage1(x, W_kqv, b_kqv.reshape(1, -1), Wk, Wq)
    g3 = _sc_pass1(src1, dst1, kW, qW)
    ex = _stage2(g3.reshape(E, EMB), edge_attr, W_edge,
                 b_edge.reshape(1, -1), We, b_a1.reshape(1, -1),
                 W_a2, b_a2.reshape(1, 1))
    Um, Us = _sc_pass2(src1, dst1, ex, v)
    out = _stage3(Um[:N], Um[N:], Us[:N, :L], Us[N:, :L],
                  x, W_out, b_out.reshape(1, -1),
                  bn_gamma.reshape(1, -1), bn_beta.reshape(1, -1),
                  bn_mean.reshape(1, -1), bn_var.reshape(1, -1))
    return out


# speedup vs baseline: 5.6050x; 1.0121x over previous
import functools
import math

import jax
import jax.numpy as jnp
from jax import lax
from jax.experimental import pallas as pl
from jax.experimental.pallas import tpu as pltpu
from jax.experimental.pallas import tpu_sc as plsc

N = 10000
E = 320000
EMB = 128
H = 2
HD = EMB // H
EA = 2

NC = 2
NS = 16
NW = NC * NS
L = 16

EPW = E // NW
CB = 80
KC = 5
C = CB * KC
NCHUNK = EPW // C
CB2 = 40
KC2 = 5
C2 = CB2 * KC2
NCHUNK2 = EPW // C2
RPT = 624
RW = EMB + L
SR = 64

_f32 = jnp.float32


def _stage1_body(x_ref, wkqv_ref, bkqv_ref, wk_ref, wq_ref,
                 kw_ref, qw_ref, v_ref):
    x = x_ref[...]
    kqv = lax.dot_general(x, wkqv_ref[...], (((1,), (1,)), ((), ())),
                          preferred_element_type=_f32) + bkqv_ref[...]
    q = kqv[:, :EMB]
    k = kqv[:, EMB:2 * EMB] * (1.0 / math.sqrt(HD))
    v_ref[...] = kqv[:, 2 * EMB:]
    kw_ref[...] = jnp.concatenate(
        [lax.dot_general(k[:, h * HD:(h + 1) * HD], wk_ref[...],
                         (((1,), (1,)), ((), ())), preferred_element_type=_f32)
         for h in range(H)], axis=1)
    qw_ref[...] = jnp.concatenate(
        [lax.dot_general(q[:, h * HD:(h + 1) * HD], wq_ref[...],
                         (((1,), (1,)), ((), ())), preferred_element_type=_f32)
         for h in range(H)], axis=1)


def _stage1(x, W_kqv, b_kqv2, Wk, Wq, interpret=False):
    BN_ = 1000
    grid = (N // BN_,)
    return pl.pallas_call(
        _stage1_body,
        grid=grid,
        in_specs=[
            pl.BlockSpec((BN_, EMB), lambda i: (i, 0)),
            pl.BlockSpec((3 * EMB, EMB), lambda i: (0, 0)),
            pl.BlockSpec((1, 3 * EMB), lambda i: (0, 0)),
            pl.BlockSpec((HD, HD), lambda i: (0, 0)),
            pl.BlockSpec((HD, HD), lambda i: (0, 0)),
        ],
        out_specs=[
            pl.BlockSpec((BN_, EMB), lambda i: (i, 0)),
            pl.BlockSpec((BN_, EMB), lambda i: (i, 0)),
            pl.BlockSpec((BN_, EMB), lambda i: (i, 0)),
        ],
        out_shape=[
            jax.ShapeDtypeStruct((N, EMB), _f32),
            jax.ShapeDtypeStruct((N, EMB), _f32),
            jax.ShapeDtypeStruct((N, EMB), _f32),
        ],
        interpret=interpret,
    )(x, W_kqv, b_kqv2, Wk, Wq)


def _sc_pass1(src1, dst1, kW, qW, interpret=False):
    mesh = plsc.VectorSubcoreMesh(core_axis_name="c", subcore_axis_name="s")

    @functools.partial(
        pl.kernel,
        out_type=jax.ShapeDtypeStruct((E // CB, CB, EMB), _f32),
        mesh=mesh,
        scratch_types=[
            pltpu.VMEM((C,), jnp.int32),
            pltpu.VMEM((C,), jnp.int32),
            pltpu.VMEM((KC, CB, EMB), _f32),
            pltpu.VMEM((KC, CB, EMB), _f32),
            pltpu.SemaphoreType.DMA,
            pltpu.SemaphoreType.DMA,
            pltpu.SemaphoreType.DMA,
        ],
        interpret=interpret,
    )
    def body(src_hbm, dst_hbm, kw_hbm, qw_hbm, g_hbm,
             sidx, didx, krows, qrows, semi, semg, semw):
        cid = lax.axis_index("c")
        sid = lax.axis_index("s")
        wid = sid * NC + cid
        rows_pw = EPW // CB

        def chunk(i, carry):
            row0 = wid * rows_pw + i * KC
            off = row0 * CB
            cpi = [pltpu.async_copy(src_hbm.at[pl.ds(off, C)], sidx, semi),
                   pltpu.async_copy(dst_hbm.at[pl.ds(off, C)], didx, semi)]
            for cp in cpi:
                cp.wait()
            cps = []
            for j in range(KC):
                cps.append(pltpu.async_copy(
                    kw_hbm.at[sidx.at[pl.ds(j * CB, CB)]], krows.at[j], semg))
                cps.append(pltpu.async_copy(
                    qw_hbm.at[didx.at[pl.ds(j * CB, CB)]], qrows.at[j], semg))
            for cp in cps:
                cp.wait()

            def add_rows(e, c2):
                for j in range(KC):
                    for t in range(EMB // L):
                        sl = pl.ds(t * L, L)
                        krows[j, e, sl] = krows[j, e, sl] + qrows[j, e, sl]
                return c2

            lax.fori_loop(0, CB, add_rows, 0, unroll=False)
            pltpu.async_copy(krows, g_hbm.at[pl.ds(row0, KC)], semw).wait()
            return carry

        lax.fori_loop(0, NCHUNK, chunk, 0, unroll=False)

    return body(src1, dst1, kW, qW)


def _stage2_body(g_ref, ea_ref, wedge_ref, bedge_ref, we_ref, ba1_ref,
                 wa2_ref, ba2_ref, ex_ref):
    ea = lax.dot_general(ea_ref[...], wedge_ref[...], (((1,), (1,)), ((), ())),
                         preferred_element_type=_f32) + bedge_ref[...]
    r = jnp.maximum(ea, 0.0)
    g = g_ref[...]
    cols = []
    for h in range(H):
        z = (g[:, h * HD:(h + 1) * HD]
             + lax.dot_general(r[:, h * HD:(h + 1) * HD], we_ref[...],
                               (((1,), (1,)), ((), ())),
                               preferred_element_type=_f32)
             + ba1_ref[...])
        z = jnp.maximum(z, 0.0)
        a_h = jnp.sum(z * wa2_ref[...], axis=1, keepdims=True) + ba2_ref[...]
        cols.append(jnp.exp(a_h))
    b = cols[0].shape[0]
    cols.append(jnp.ones((b, 1), _f32))
    cols.append(jnp.zeros((b, L - H - 1), _f32))
    ex_ref[...] = jnp.concatenate(cols, axis=1)


def _stage2(g, edge_attr, W_edge, b_edge2, We, b_a12, W_a2, b_a22,
            interpret=False):
    BE = 4000
    grid = (E // BE,)
    return pl.pallas_call(
        _stage2_body,
        grid=grid,
        in_specs=[
            pl.BlockSpec((BE, EMB), lambda i: (i, 0)),
            pl.BlockSpec((BE, EA), lambda i: (i, 0)),
            pl.BlockSpec((EMB, EA), lambda i: (0, 0)),
            pl.BlockSpec((1, EMB), lambda i: (0, 0)),
            pl.BlockSpec((HD, HD), lambda i: (0, 0)),
            pl.BlockSpec((1, HD), lambda i: (0, 0)),
            pl.BlockSpec((1, HD), lambda i: (0, 0)),
            pl.BlockSpec((1, 1), lambda i: (0, 0)),
        ],
        out_specs=pl.BlockSpec((BE, L), lambda i: (i, 0)),
        out_shape=jax.ShapeDtypeStruct((E, L), _f32),
        interpret=interpret,
    )(g, edge_attr, W_edge, b_edge2, We, b_a12, W_a2, b_a22)


def _sc_pass2(src1, dst1, ex2, v, interpret=False):
    mesh = plsc.VectorSubcoreMesh(core_axis_name="c", subcore_axis_name="s")

    @functools.partial(
        pl.kernel,
        out_type=[
            jax.ShapeDtypeStruct((NC * N, EMB), _f32),
            jax.ShapeDtypeStruct((NC * N, EMB), _f32),
        ],
        mesh=mesh,
        scratch_types=[
            pltpu.VMEM((C2,), jnp.int32),
            pltpu.VMEM((KC2, CB2), jnp.int32),
            pltpu.VMEM((2, CB2, L), _f32),
            pltpu.VMEM((KC2, CB2, EMB), _f32),
            pltpu.VMEM_SHARED((N, EMB), _f32),
            pltpu.SemaphoreType.DMA,
            pltpu.SemaphoreType.DMA,
            pltpu.SemaphoreType.DMA,
            pltpu.SemaphoreType.DMA,
            pltpu.SemaphoreType.DMA,
        ],
        interpret=interpret,
    )
    def body(src_hbm, dst_hbm, ex_hbm, v_hbm, zrow_hbm, u_hbm, s_hbm,
             sidx, didx, exb, vrows, u_sh, semi, semg, seme0, seme1, sems):
        cid = lax.axis_index("c")
        sid = lax.axis_index("s")
        wid = sid * NC + cid
        rows_pw = EPW // CB2

        r0 = sid * RPT
        t0 = jnp.where(sid == 0, NS * RPT, r0)
        tail = N - NS * RPT

        def zero_acc():
            pltpu.sync_copy(zrow_hbm.at[pl.ds(r0, RPT)],
                            u_sh.at[pl.ds(r0, RPT)])
            pltpu.sync_copy(zrow_hbm.at[pl.ds(t0, tail)],
                            u_sh.at[pl.ds(t0, tail)])

        def read_acc(out_hbm):
            pltpu.sync_copy(u_sh.at[pl.ds(r0, RPT)],
                            out_hbm.at[pl.ds(cid * N + r0, RPT)])
            pltpu.sync_copy(u_sh.at[pl.ds(t0, tail)],
                            out_hbm.at[pl.ds(cid * N + t0, tail)])

        def load_idx_async(off, with_src):
            cps = []
            if with_src:
                cps.append(pltpu.async_copy(src_hbm.at[pl.ds(off, C2)],
                                            sidx, semi))
            for j in range(KC2):
                cps.append(pltpu.async_copy(
                    dst_hbm.at[pl.ds(off + j * CB2, CB2)], didx.at[j], semi))
            return cps

        seme = [seme0, seme1]

        def fire_exb(off, j):
            return pltpu.async_copy(
                ex_hbm.at[pl.ds(off + j * CB2, CB2)], exb.at[j % 2],
                seme[j % 2])

        zero_acc()
        plsc.subcore_barrier()

        def chunk_a(i, carry):
            off = (wid * rows_pw + i * KC2) * CB2
            cpi = load_idx_async(off, with_src=True)
            cpe = {0: fire_exb(off, 0)}
            for cp in cpi:
                cp.wait()
            cpg = [pltpu.async_copy(
                       v_hbm.at[sidx.at[pl.ds(j * CB2, CB2)]], vrows.at[j],
                       semg)
                   for j in range(KC2)]
            for cp in cpg:
                cp.wait()
            cps = []
            for j in range(KC2):
                if j + 1 < KC2:
                    cpe[j + 1] = fire_exb(off, j + 1)
                cpe[j].wait()
                b = j % 2

                def scale_rows(e, c2):
                    s = exb[b, e, :]
                    e0v = jnp.broadcast_to(s[0], (L,))
                    e1v = jnp.broadcast_to(s[1], (L,))
                    for t in range(HD // L):
                        sl = pl.ds(t * L, L)
                        vrows[j, e, sl] = vrows[j, e, sl] * e0v
                    for t in range(HD // L, EMB // L):
                        sl = pl.ds(t * L, L)
                        vrows[j, e, sl] = vrows[j, e, sl] * e1v
                    return c2

                lax.fori_loop(0, CB2, scale_rows, 0, unroll=False)
                cps.append(pltpu.async_copy(vrows.at[j],
                                            u_sh.at[didx.at[j]], sems,
                                            add=True))
            for cp in cps:
                cp.wait()
            return carry

        lax.fori_loop(0, NCHUNK2, chunk_a, 0, unroll=False)
        plsc.subcore_barrier()
        read_acc(u_hbm)
        plsc.subcore_barrier()

        zero_acc()
        zero16 = jnp.zeros((L,), _f32)

        def zero_vrows(e, c2):
            for j in range(KC2):
                for t in range(EMB // L):
                    vrows[j, e, pl.ds(t * L, L)] = zero16
            return c2

        lax.fori_loop(0, CB2, zero_vrows, 0, unroll=False)
        plsc.subcore_barrier()

        def chunk_b(i, carry):
            off = (wid * rows_pw + i * KC2) * CB2
            cpi = load_idx_async(off, with_src=False)
            cpe = {0: fire_exb(off, 0)}
            for cp in cpi:
                cp.wait()
            cps = []
            for j in range(KC2):
                if j + 1 < KC2:
                    cpe[j + 1] = fire_exb(off, j + 1)
                cpe[j].wait()
                b = j % 2

                def stat_rows(e, c2):
                    vrows[j, e, pl.ds(0, L)] = exb[b, e, :]
                    return c2

                lax.fori_loop(0, CB2, stat_rows, 0, unroll=False)
                cps.append(pltpu.async_copy(vrows.at[j],
                                            u_sh.at[didx.at[j]], sems,
                                            add=True))
            for cp in cps:
                cp.wait()
            return carry

        lax.fori_loop(0, NCHUNK2, chunk_b, 0, unroll=False)
        plsc.subcore_barrier()
        read_acc(s_hbm)

    zrow = jnp.zeros((N, EMB), _f32)
    return body(src1, dst1, ex2, v, zrow)


def _stage3_body(u0_ref, u1_ref, s0_ref, s1_ref, x_ref, wout_ref, bout_ref,
                 gam_ref, bet_ref, mu_ref, var_ref, out_ref):
    um = u0_ref[...] + u1_ref[...]
    us = s0_ref[...] + s1_ref[...]
    d0 = us[:, 0:1] + 1e-16
    d1 = us[:, 1:2] + 1e-16
    indeg = us[:, 2:3]
    aggp = jnp.concatenate([um[:, :HD] / d0, um[:, HD:] / d1], axis=1)
    agg = lax.dot_general(aggp, wout_ref[...], (((1,), (1,)), ((), ())),
                          preferred_element_type=_f32) + indeg * bout_ref[...]
    o = agg + x_ref[...]
    o = (o - mu_ref[...]) * lax.rsqrt(var_ref[...] + 1e-5) * gam_ref[...] \
        + bet_ref[...]
    out_ref[...] = o


def _stage3(U0, U1, S0, S1, x, W_out, b_out2, gam2, bet2, mu2, var2,
            interpret=False):
    BN_ = 1000
    grid = (N // BN_,)
    return pl.pallas_call(
        _stage3_body,
        grid=grid,
        in_specs=[
            pl.BlockSpec((BN_, EMB), lambda i: (i, 0)),
            pl.BlockSpec((BN_, EMB), lambda i: (i, 0)),
            pl.BlockSpec((BN_, L), lambda i: (i, 0)),
            pl.BlockSpec((BN_, L), lambda i: (i, 0)),
            pl.BlockSpec((BN_, EMB), lambda i: (i, 0)),
            pl.BlockSpec((EMB, EMB), lambda i: (0, 0)),
            pl.BlockSpec((1, EMB), lambda i: (0, 0)),
            pl.BlockSpec((1, EMB), lambda i: (0, 0)),
            pl.BlockSpec((1, EMB), lambda i: (0, 0)),
            pl.BlockSpec((1, EMB), lambda i: (0, 0)),
            pl.BlockSpec((1, EMB), lambda i: (0, 0)),
        ],
        out_specs=pl.BlockSpec((BN_, EMB), lambda i: (i, 0)),
        out_shape=jax.ShapeDtypeStruct((N, EMB), _f32),
        interpret=interpret,
    )(U0, U1, S0, S1, x, W_out, b_out2, gam2, bet2, mu2, var2)


def kernel(x, edge_index, edge_attr, W_kqv, b_kqv, W_edge, b_edge,
           W_a1, b_a1, W_a2, b_a2, W_out, b_out,
           bn_gamma, bn_beta, bn_mean, bn_var):
    src1 = edge_index[0]
    dst1 = edge_index[1]
    Wk = W_a1[:, :HD]
    Wq = W_a1[:, HD:2 * HD]
    We = W_a1[:, 2 * HD:]

    kW, qW, v = _stage1(x, W_kqv, b_kqv.reshape(1, -1), Wk, Wq)
    g3 = _sc_pass1(src1, dst1, kW, qW)
    ex = _stage2(g3.reshape(E, EMB), edge_attr, W_edge,
                 b_edge.reshape(1, -1), We, b_a1.reshape(1, -1),
                 W_a2, b_a2.reshape(1, 1))
    Um, Us = _sc_pass2(src1, dst1, ex, v)
    out = _stage3(Um[:N], Um[N:], Us[:N, :L], Us[N:, :L],
                  x, W_out, b_out.reshape(1, -1),
                  bn_gamma.reshape(1, -1), bn_beta.reshape(1, -1),
                  bn_mean.reshape(1, -1), bn_var.reshape(1, -1))
    return out


# speedup vs baseline: 5.6378x; 1.0058x over previous
import functools
import math

import jax
import jax.numpy as jnp
from jax import lax
from jax.experimental import pallas as pl
from jax.experimental.pallas import tpu as pltpu
from jax.experimental.pallas import tpu_sc as plsc

N = 10000
E = 320000
EMB = 128
H = 2
HD = EMB // H
EA = 2

NC = 2
NS = 16
NW = NC * NS
L = 16

EPW = E // NW
CB = 80
KC = 5
C = CB * KC
NCHUNK = EPW // C
CB2 = 40
KC2 = 5
C2 = CB2 * KC2
NCHUNK2 = EPW // C2
RPT = 624
RW = EMB + L
SR = 64

_f32 = jnp.float32


def _stage1_body(x_ref, wkqv_ref, bkqv_ref, wk_ref, wq_ref,
                 kw_ref, qw_ref, v_ref):
    x = x_ref[...]
    kqv = lax.dot_general(x, wkqv_ref[...], (((1,), (1,)), ((), ())),
                          preferred_element_type=_f32) + bkqv_ref[...]
    q = kqv[:, :EMB]
    k = kqv[:, EMB:2 * EMB] * (1.0 / math.sqrt(HD))
    v_ref[...] = kqv[:, 2 * EMB:]
    kw_ref[...] = jnp.concatenate(
        [lax.dot_general(k[:, h * HD:(h + 1) * HD], wk_ref[...],
                         (((1,), (1,)), ((), ())), preferred_element_type=_f32)
         for h in range(H)], axis=1)
    qw_ref[...] = jnp.concatenate(
        [lax.dot_general(q[:, h * HD:(h + 1) * HD], wq_ref[...],
                         (((1,), (1,)), ((), ())), preferred_element_type=_f32)
         for h in range(H)], axis=1)


def _stage1(x, W_kqv, b_kqv2, Wk, Wq, interpret=False):
    BN_ = 1000
    grid = (N // BN_,)
    return pl.pallas_call(
        _stage1_body,
        grid=grid,
        in_specs=[
            pl.BlockSpec((BN_, EMB), lambda i: (i, 0)),
            pl.BlockSpec((3 * EMB, EMB), lambda i: (0, 0)),
            pl.BlockSpec((1, 3 * EMB), lambda i: (0, 0)),
            pl.BlockSpec((HD, HD), lambda i: (0, 0)),
            pl.BlockSpec((HD, HD), lambda i: (0, 0)),
        ],
        out_specs=[
            pl.BlockSpec((BN_, EMB), lambda i: (i, 0)),
            pl.BlockSpec((BN_, EMB), lambda i: (i, 0)),
            pl.BlockSpec((BN_, EMB), lambda i: (i, 0)),
        ],
        out_shape=[
            jax.ShapeDtypeStruct((N, EMB), _f32),
            jax.ShapeDtypeStruct((N, EMB), _f32),
            jax.ShapeDtypeStruct((N, EMB), _f32),
        ],
        interpret=interpret,
    )(x, W_kqv, b_kqv2, Wk, Wq)


def _sc_pass1(src1, dst1, kW, qW, interpret=False):
    mesh = plsc.VectorSubcoreMesh(core_axis_name="c", subcore_axis_name="s")

    @functools.partial(
        pl.kernel,
        out_type=jax.ShapeDtypeStruct((E // CB, CB, EMB), _f32),
        mesh=mesh,
        scratch_types=[
            pltpu.VMEM((C,), jnp.int32),
            pltpu.VMEM((C,), jnp.int32),
            pltpu.VMEM((KC, CB, EMB), _f32),
            pltpu.VMEM((KC, CB, EMB), _f32),
            pltpu.SemaphoreType.DMA,
            pltpu.SemaphoreType.DMA,
            pltpu.SemaphoreType.DMA,
            pltpu.SemaphoreType.DMA,
            pltpu.SemaphoreType.DMA,
            pltpu.SemaphoreType.DMA,
            pltpu.SemaphoreType.DMA,
        ],
        interpret=interpret,
    )
    def body(src_hbm, dst_hbm, kw_hbm, qw_hbm, g_hbm,
             sidx, didx, krows, qrows, semi, semw,
             sg0, sg1, sg2, sg3, sg4):
        cid = lax.axis_index("c")
        sid = lax.axis_index("s")
        wid = sid * NC + cid
        rows_pw = EPW // CB

        def chunk(i, carry):
            row0 = wid * rows_pw + i * KC
            off = row0 * CB
            cpi = [pltpu.async_copy(src_hbm.at[pl.ds(off, C)], sidx, semi),
                   pltpu.async_copy(dst_hbm.at[pl.ds(off, C)], didx, semi)]
            for cp in cpi:
                cp.wait()
            sgs = [sg0, sg1, sg2, sg3, sg4]
            cps = []
            for j in range(KC):
                cps.append((
                    pltpu.async_copy(kw_hbm.at[sidx.at[pl.ds(j * CB, CB)]],
                                     krows.at[j], sgs[j]),
                    pltpu.async_copy(qw_hbm.at[didx.at[pl.ds(j * CB, CB)]],
                                     qrows.at[j], sgs[j])))
            for j in range(KC):
                cps[j][0].wait()
                cps[j][1].wait()

                def add_rows(e, c2):
                    for t in range(EMB // L):
                        sl = pl.ds(t * L, L)
                        krows[j, e, sl] = krows[j, e, sl] + qrows[j, e, sl]
                    return c2

                lax.fori_loop(0, CB, add_rows, 0, unroll=False)
            pltpu.async_copy(krows, g_hbm.at[pl.ds(row0, KC)], semw).wait()
            return carry

        lax.fori_loop(0, NCHUNK, chunk, 0, unroll=False)

    return body(src1, dst1, kW, qW)


def _stage2_body(g_ref, ea_ref, wedge_ref, bedge_ref, we_ref, ba1_ref,
                 wa2_ref, ba2_ref, ex_ref):
    ea = lax.dot_general(ea_ref[...], wedge_ref[...], (((1,), (1,)), ((), ())),
                         preferred_element_type=_f32) + bedge_ref[...]
    r = jnp.maximum(ea, 0.0)
    g = g_ref[...]
    cols = []
    for h in range(H):
        z = (g[:, h * HD:(h + 1) * HD]
             + lax.dot_general(r[:, h * HD:(h + 1) * HD], we_ref[...],
                               (((1,), (1,)), ((), ())),
                               preferred_element_type=_f32)
             + ba1_ref[...])
        z = jnp.maximum(z, 0.0)
        a_h = jnp.sum(z * wa2_ref[...], axis=1, keepdims=True) + ba2_ref[...]
        cols.append(jnp.exp(a_h))
    b = cols[0].shape[0]
    cols.append(jnp.ones((b, 1), _f32))
    cols.append(jnp.zeros((b, L - H - 1), _f32))
    ex_ref[...] = jnp.concatenate(cols, axis=1)


def _stage2(g, edge_attr, W_edge, b_edge2, We, b_a12, W_a2, b_a22,
            interpret=False):
    BE = 4000
    grid = (E // BE,)
    return pl.pallas_call(
        _stage2_body,
        grid=grid,
        in_specs=[
            pl.BlockSpec((BE, EMB), lambda i: (i, 0)),
            pl.BlockSpec((BE, EA), lambda i: (i, 0)),
            pl.BlockSpec((EMB, EA), lambda i: (0, 0)),
            pl.BlockSpec((1, EMB), lambda i: (0, 0)),
            pl.BlockSpec((HD, HD), lambda i: (0, 0)),
            pl.BlockSpec((1, HD), lambda i: (0, 0)),
            pl.BlockSpec((1, HD), lambda i: (0, 0)),
            pl.BlockSpec((1, 1), lambda i: (0, 0)),
        ],
        out_specs=pl.BlockSpec((BE, L), lambda i: (i, 0)),
        out_shape=jax.ShapeDtypeStruct((E, L), _f32),
        interpret=interpret,
    )(g, edge_attr, W_edge, b_edge2, We, b_a12, W_a2, b_a22)


def _sc_pass2(src1, dst1, ex2, v, interpret=False):
    mesh = plsc.VectorSubcoreMesh(core_axis_name="c", subcore_axis_name="s")

    @functools.partial(
        pl.kernel,
        out_type=[
            jax.ShapeDtypeStruct((NC * N, EMB), _f32),
            jax.ShapeDtypeStruct((NC * N, EMB), _f32),
        ],
        mesh=mesh,
        scratch_types=[
            pltpu.VMEM((C2,), jnp.int32),
            pltpu.VMEM((KC2, CB2), jnp.int32),
            pltpu.VMEM((2, CB2, L), _f32),
            pltpu.VMEM((KC2, CB2, EMB), _f32),
            pltpu.VMEM_SHARED((N, EMB), _f32),
            pltpu.SemaphoreType.DMA,
            pltpu.SemaphoreType.DMA,
            pltpu.SemaphoreType.DMA,
            pltpu.SemaphoreType.DMA,
            pltpu.SemaphoreType.DMA,
            pltpu.SemaphoreType.DMA,
            pltpu.SemaphoreType.DMA,
            pltpu.SemaphoreType.DMA,
            pltpu.SemaphoreType.DMA,
        ],
        interpret=interpret,
    )
    def body(src_hbm, dst_hbm, ex_hbm, v_hbm, zrow_hbm, u_hbm, s_hbm,
             sidx, didx, exb, vrows, u_sh, semi, seme0, seme1, sems,
             sg0, sg1, sg2, sg3, sg4):
        cid = lax.axis_index("c")
        sid = lax.axis_index("s")
        wid = sid * NC + cid
        rows_pw = EPW // CB2

        r0 = sid * RPT
        t0 = jnp.where(sid == 0, NS * RPT, r0)
        tail = N - NS * RPT

        def zero_acc():
            pltpu.sync_copy(zrow_hbm.at[pl.ds(r0, RPT)],
                            u_sh.at[pl.ds(r0, RPT)])
            pltpu.sync_copy(zrow_hbm.at[pl.ds(t0, tail)],
                            u_sh.at[pl.ds(t0, tail)])

        def read_acc(out_hbm):
            pltpu.sync_copy(u_sh.at[pl.ds(r0, RPT)],
                            out_hbm.at[pl.ds(cid * N + r0, RPT)])
            pltpu.sync_copy(u_sh.at[pl.ds(t0, tail)],
                            out_hbm.at[pl.ds(cid * N + t0, tail)])

        def load_idx_async(off, with_src):
            cps = []
            if with_src:
                cps.append(pltpu.async_copy(src_hbm.at[pl.ds(off, C2)],
                                            sidx, semi))
            for j in range(KC2):
                cps.append(pltpu.async_copy(
                    dst_hbm.at[pl.ds(off + j * CB2, CB2)], didx.at[j], semi))
            return cps

        seme = [seme0, seme1]

        def fire_exb(off, j):
            return pltpu.async_copy(
                ex_hbm.at[pl.ds(off + j * CB2, CB2)], exb.at[j % 2],
                seme[j % 2])

        zero_acc()
        plsc.subcore_barrier()

        def chunk_a(i, carry):
            off = (wid * rows_pw + i * KC2) * CB2
            sgs = [sg0, sg1, sg2, sg3, sg4]
            cpi = load_idx_async(off, with_src=True)
            cpe = {0: fire_exb(off, 0)}
            for cp in cpi:
                cp.wait()
            cpg = [pltpu.async_copy(
                       v_hbm.at[sidx.at[pl.ds(j * CB2, CB2)]], vrows.at[j],
                       sgs[j])
                   for j in range(KC2)]
            cps = []
            for j in range(KC2):
                if j + 1 < KC2:
                    cpe[j + 1] = fire_exb(off, j + 1)
                cpg[j].wait()
                cpe[j].wait()
                b = j % 2

                def scale_rows(e, c2):
                    s = exb[b, e, :]
                    e0v = jnp.broadcast_to(s[0], (L,))
                    e1v = jnp.broadcast_to(s[1], (L,))
                    for t in range(HD // L):
                        sl = pl.ds(t * L, L)
                        vrows[j, e, sl] = vrows[j, e, sl] * e0v
                    for t in range(HD // L, EMB // L):
                        sl = pl.ds(t * L, L)
                        vrows[j, e, sl] = vrows[j, e, sl] * e1v
                    return c2

                lax.fori_loop(0, CB2, scale_rows, 0, unroll=False)
                cps.append(pltpu.async_copy(vrows.at[j],
                                            u_sh.at[didx.at[j]], sems,
                                            add=True))
            for cp in cps:
                cp.wait()
            return carry

        lax.fori_loop(0, NCHUNK2, chunk_a, 0, unroll=False)
        plsc.subcore_barrier()
        read_acc(u_hbm)
        plsc.subcore_barrier()

        zero_acc()
        zero16 = jnp.zeros((L,), _f32)

        def zero_vrows(e, c2):
            for j in range(KC2):
                for t in range(EMB // L):
                    vrows[j, e, pl.ds(t * L, L)] = zero16
            return c2

        lax.fori_loop(0, CB2, zero_vrows, 0, unroll=False)
        plsc.subcore_barrier()

        def chunk_b(i, carry):
            off = (wid * rows_pw + i * KC2) * CB2
            cpi = load_idx_async(off, with_src=False)
            cpe = {0: fire_exb(off, 0)}
            for cp in cpi:
                cp.wait()
            cps = []
            for j in range(KC2):
                if j + 1 < KC2:
                    cpe[j + 1] = fire_exb(off, j + 1)
                cpe[j].wait()
                b = j % 2

                def stat_rows(e, c2):
                    vrows[j, e, pl.ds(0, L)] = exb[b, e, :]
                    return c2

                lax.fori_loop(0, CB2, stat_rows, 0, unroll=False)
                cps.append(pltpu.async_copy(vrows.at[j],
                                            u_sh.at[didx.at[j]], sems,
                                            add=True))
            for cp in cps:
                cp.wait()
            return carry

        lax.fori_loop(0, NCHUNK2, chunk_b, 0, unroll=False)
        plsc.subcore_barrier()
        read_acc(s_hbm)

    zrow = jnp.zeros((N, EMB), _f32)
    return body(src1, dst1, ex2, v, zrow)


def _stage3_body(u0_ref, u1_ref, s0_ref, s1_ref, x_ref, wout_ref, bout_ref,
                 gam_ref, bet_ref, mu_ref, var_ref, out_ref):
    um = u0_ref[...] + u1_ref[...]
    us = s0_ref[...] + s1_ref[...]
    d0 = us[:, 0:1] + 1e-16
    d1 = us[:, 1:2] + 1e-16
    indeg = us[:, 2:3]
    aggp = jnp.concatenate([um[:, :HD] / d0, um[:, HD:] / d1], axis=1)
    agg = lax.dot_general(aggp, wout_ref[...], (((1,), (1,)), ((), ())),
                          preferred_element_type=_f32) + indeg * bout_ref[...]
    o = agg + x_ref[...]
    o = (o - mu_ref[...]) * lax.rsqrt(var_ref[...] + 1e-5) * gam_ref[...] \
        + bet_ref[...]
    out_ref[...] = o


def _stage3(U0, U1, S0, S1, x, W_out, b_out2, gam2, bet2, mu2, var2,
            interpret=False):
    BN_ = 1000
    grid = (N // BN_,)
    return pl.pallas_call(
        _stage3_body,
        grid=grid,
        in_specs=[
            pl.BlockSpec((BN_, EMB), lambda i: (i, 0)),
            pl.BlockSpec((BN_, EMB), lambda i: (i, 0)),
            pl.BlockSpec((BN_, L), lambda i: (i, 0)),
            pl.BlockSpec((BN_, L), lambda i: (i, 0)),
            pl.BlockSpec((BN_, EMB), lambda i: (i, 0)),
            pl.BlockSpec((EMB, EMB), lambda i: (0, 0)),
            pl.BlockSpec((1, EMB), lambda i: (0, 0)),
            pl.BlockSpec((1, EMB), lambda i: (0, 0)),
            pl.BlockSpec((1, EMB), lambda i: (0, 0)),
            pl.BlockSpec((1, EMB), lambda i: (0, 0)),
            pl.BlockSpec((1, EMB), lambda i: (0, 0)),
        ],
        out_specs=pl.BlockSpec((BN_, EMB), lambda i: (i, 0)),
        out_shape=jax.ShapeDtypeStruct((N, EMB), _f32),
        interpret=interpret,
    )(U0, U1, S0, S1, x, W_out, b_out2, gam2, bet2, mu2, var2)


def kernel(x, edge_index, edge_attr, W_kqv, b_kqv, W_edge, b_edge,
           W_a1, b_a1, W_a2, b_a2, W_out, b_out,
           bn_gamma, bn_beta, bn_mean, bn_var):
    src1 = edge_index[0]
    dst1 = edge_index[1]
    Wk = W_a1[:, :HD]
    Wq = W_a1[:, HD:2 * HD]
    We = W_a1[:, 2 * HD:]

    kW, qW, v = _stage1(x, W_kqv, b_kqv.reshape(1, -1), Wk, Wq)
    g3 = _sc_pass1(src1, dst1, kW, qW)
    ex = _stage2(g3.reshape(E, EMB), edge_attr, W_edge,
                 b_edge.reshape(1, -1), We, b_a1.reshape(1, -1),
                 W_a2, b_a2.reshape(1, 1))
    Um, Us = _sc_pass2(src1, dst1, ex, v)
    out = _stage3(Um[:N], Um[N:], Us[:N, :L], Us[N:, :L],
                  x, W_out, b_out.reshape(1, -1),
                  bn_gamma.reshape(1, -1), bn_beta.reshape(1, -1),
                  bn_mean.reshape(1, -1), bn_var.reshape(1, -1))
    return out
